# Initial kernel scaffold; baseline (speedup 1.0000x reference)
import jax
import jax.numpy as jnp
from jax.experimental import pallas as pl


def kernel(x, W, gamma, beta):
    raise NotImplementedError("write your pallas kernel here")


# speedup vs baseline: 13.2594x; 13.2594x over previous
import functools

import jax
import jax.numpy as jnp
from jax.experimental import pallas as pl
from jax.experimental.pallas import tpu as pltpu

_K = 20
_EPS = 1e-5
_SLOPE = 0.2
_NEG = -3.0e38


def _edge_kernel(xt_rows_ref, x_full_ref, xt_full_ref, w1t_ref, wd_ref,
                 pmax_ref, pmin_ref, spart_ref, qpart_ref, gt_ref):
    t = pl.program_id(1)

    x_b = x_full_ref[0]
    rows = xt_rows_ref[0]
    R = rows.shape[0]
    N = x_b.shape[1]

    @pl.when(t == 0)
    def _():
        gt_ref[...] = jax.lax.dot_general(
            xt_full_ref[0], w1t_ref[...], (((1,), (0,)), ((), ())),
            preferred_element_type=jnp.float32)

    gt = gt_ref[...]

    d2 = jax.lax.dot_general(rows, x_b, (((1,), (0,)), ((), ())),
                             preferred_element_type=jnp.float32)
    xxr = jnp.sum(rows * rows, axis=1, keepdims=True)
    xxc = jnp.sum(x_b * x_b, axis=0, keepdims=True)
    negd = 2.0 * d2 - xxr - xxc

    lane = jax.lax.broadcasted_iota(jnp.int32, (R, N), 1)

    ht = jax.lax.dot_general(rows, wd_ref[...], (((1,), (0,)), ((), ())),
                             preferred_element_type=jnp.float32)

    maxv = None
    for j in range(_K):
        m = jnp.max(negd, axis=1, keepdims=True)
        cand = jnp.where(negd == m, lane, N)
        am = jnp.min(cand, axis=1, keepdims=True)
        onehot = (lane == am)
        v = jax.lax.dot_general(onehot.astype(jnp.float32), gt,
                                (((1,), (0,)), ((), ())),
                                preferred_element_type=jnp.float32)
        negd = jnp.where(onehot, _NEG, negd)
        if maxv is None:
            maxv, minv, sumv, sumsq = v, v, v, v * v
        else:
            maxv = jnp.maximum(maxv, v)
            minv = jnp.minimum(minv, v)
            sumv = sumv + v
            sumsq = sumsq + v * v

    pmax_ref[...] = (maxv + ht)[None]
    pmin_ref[...] = (minv + ht)[None]
    kf = jnp.float32(_K)
    spart = jnp.sum(sumv + kf * ht, axis=0, keepdims=True)
    qpart = jnp.sum(sumsq + 2.0 * ht * sumv + kf * ht * ht, axis=0,
                    keepdims=True)
    spart_ref[...] = spart[None]
    qpart_ref[...] = qpart[None]


def _bn_kernel(pmax_ref, pmin_ref, scale_ref, shift_ref, out_ref):
    scale = scale_ref[...]
    shift = shift_ref[...]
    pick = jnp.where(scale >= 0.0, pmax_ref[0], pmin_ref[0])
    z = pick * scale + shift
    out_ref[...] = jnp.where(z >= 0.0, z, _SLOPE * z)[None]


@jax.jit
def kernel(x, W, gamma, beta):
    B, C, N = x.shape
    Co = W.shape[0]
    R = 256
    T = N // R
    G = B * T

    xt = jnp.transpose(x, (0, 2, 1))
    w1t = jnp.transpose(W[:, :C])
    wd = jnp.transpose(W[:, C:] - W[:, :C])

    grid = (B, T)
    pmax, pmin, spart, qpart = pl.pallas_call(
        _edge_kernel,
        grid=grid,
        in_specs=[
            pl.BlockSpec((1, R, C), lambda b, t: (b, t, 0)),
            pl.BlockSpec((1, C, N), lambda b, t: (b, 0, 0)),
            pl.BlockSpec((1, N, C), lambda b, t: (b, 0, 0)),
            pl.BlockSpec((C, Co), lambda b, t: (0, 0)),
            pl.BlockSpec((C, Co), lambda b, t: (0, 0)),
        ],
        out_specs=[
            pl.BlockSpec((1, R, Co), lambda b, t: (b, t, 0)),
            pl.BlockSpec((1, R, Co), lambda b, t: (b, t, 0)),
            pl.BlockSpec((1, 1, Co), lambda b, t: (b * T + t, 0, 0)),
            pl.BlockSpec((1, 1, Co), lambda b, t: (b * T + t, 0, 0)),
        ],
        out_shape=[
            jax.ShapeDtypeStruct((B, N, Co), jnp.float32),
            jax.ShapeDtypeStruct((B, N, Co), jnp.float32),
            jax.ShapeDtypeStruct((G, 1, Co), jnp.float32),
            jax.ShapeDtypeStruct((G, 1, Co), jnp.float32),
        ],
        scratch_shapes=[pltpu.VMEM((N, Co), jnp.float32)],
    )(xt, x, xt, w1t, wd)

    cnt = jnp.float32(B * N * _K)
    S = jnp.sum(spart, axis=(0, 1))
    Q = jnp.sum(qpart, axis=(0, 1))
    mean = S / cnt
    var = Q / cnt - mean * mean
    scale = gamma / jnp.sqrt(var + _EPS)
    shift = beta - scale * mean

    out_nc = pl.pallas_call(
        _bn_kernel,
        grid=(B,),
        in_specs=[
            pl.BlockSpec((1, N, Co), lambda b: (b, 0, 0)),
            pl.BlockSpec((1, N, Co), lambda b: (b, 0, 0)),
            pl.BlockSpec((1, Co), lambda b: (0, 0)),
            pl.BlockSpec((1, Co), lambda b: (0, 0)),
        ],
        out_specs=pl.BlockSpec((1, N, Co), lambda b: (b, 0, 0)),
        out_shape=jax.ShapeDtypeStruct((B, N, Co), jnp.float32),
    )(pmax, pmin, scale[None], shift[None])

    return jnp.transpose(out_nc, (0, 2, 1))
